# baseline (speedup 1.0000x reference)
import functools

import jax
import jax.numpy as jnp
from jax import lax
from jax.experimental import pallas as pl
from jax.experimental.pallas import tpu as pltpu
from jax.experimental.pallas import tpu_sc as plsc

_SCALING = 2.0
_BV = 4096
_CH = 128


def _fuse_body(w_ref, a_ref, bt_ref, o_ref):
    o_ref[...] = w_ref[...] + lax.dot_general(
        a_ref[...], bt_ref[...],
        dimension_numbers=(((0,), (0,)), ((), ())),
        preferred_element_type=jnp.float32)


def _fused_table(weight, lora_A, lora_bt2, interpret=False):
    V, D = weight.shape
    R = lora_A.shape[0]
    return pl.pallas_call(
        _fuse_body,
        grid=((V + _BV - 1) // _BV,),
        in_specs=[
            pl.BlockSpec((_BV, D), lambda i: (i, 0)),
            pl.BlockSpec((R, _BV), lambda i: (0, i)),
            pl.BlockSpec((R, D), lambda i: (0, 0)),
        ],
        out_specs=pl.BlockSpec((_BV, D), lambda i: (i, 0)),
        out_shape=jax.ShapeDtypeStruct((V, D), jnp.float32),
        interpret=interpret,
    )(weight, lora_A, lora_bt2)


_SBC = 2
_SB = _SBC * _CH


def _make_gather(n_tok, D):
    info = plsc.get_sparse_core_info()
    NC, NS = info.num_cores, info.num_subcores
    NW = NC * NS
    per_w = n_tok // NW
    n_ch = per_w // _CH
    n_sb = per_w // _SB
    mesh = plsc.VectorSubcoreMesh(core_axis_name="c", subcore_axis_name="s")

    @functools.partial(
        pl.kernel, mesh=mesh,
        out_type=jax.ShapeDtypeStruct((n_tok, D), jnp.float32),
        scratch_types=[
            pltpu.VMEM((n_ch, _CH), jnp.int32),
            pltpu.VMEM((2, _SB, D // 2), jnp.int32),
            pltpu.VMEM((2, _SB, D), jnp.float32),
        ] + [pltpu.SemaphoreType.DMA] * 4,
        compiler_params=pltpu.CompilerParams(use_tc_tiling_on_sc=False),
    )
    def gather_k(table_hbm, idx_hbm, out_hbm, idx_v, brows_v, rows_v, *sems):
        gsem = sems[:2]
        wsem = sems[2:]
        wid = lax.axis_index("s") * NC + lax.axis_index("c")
        base = wid * per_w
        pltpu.sync_copy(idx_hbm.at[pl.ds(wid * n_ch, n_ch)], idx_v)

        def start_gathers(i, b):
            for q in range(_SBC):
                pltpu.async_copy(
                    table_hbm.at[idx_v.at[i * _SBC + q]],
                    brows_v.at[b].at[pl.ds(q * _CH, _CH)], gsem[b])

        def wait_gathers(i, b):
            for q in range(_SBC):
                pltpu.make_async_copy(
                    table_hbm.at[idx_v.at[i * _SBC + q]],
                    brows_v.at[b].at[pl.ds(q * _CH, _CH)], gsem[b]).wait()

        def start_wb(i, b):
            pltpu.async_copy(
                rows_v.at[b], out_hbm.at[pl.ds(base + i * _SB, _SB)], wsem[b])

        def wait_wb(b):
            pltpu.make_async_copy(
                rows_v.at[b], out_hbm.at[pl.ds(base, _SB)], wsem[b]).wait()

        start_gathers(0, 0)

        def outer(o, carry):
            for b in range(2):
                i = o * 2 + b
                wait_gathers(i, b)

                @pl.when(i + 1 < n_sb)
                def _():
                    start_gathers(i + 1, 1 - b)
            return carry

        lax.fori_loop(0, n_sb // 2, outer, 0)
        start_wb(0, 0)
        wait_wb(0)

    return gather_k


def kernel(x, weight, lora_A, lora_B):
    B, L = x.shape
    V, D = weight.shape
    n_tok = B * L
    lora_bt2 = (_SCALING * lora_B).T
    table = _fused_table(weight, lora_A, lora_bt2).astype(jnp.bfloat16)
    table = jax.lax.bitcast_convert_type(
        table.reshape(V, D // 2, 2), jnp.int32)
    idx2d = x.reshape(n_tok // _CH, _CH).astype(jnp.int32)
    out = _make_gather(n_tok, D)(table, idx2d)
    return out.reshape(B, L, D)


# speedup vs baseline: 1.8454x; 1.8454x over previous
import functools

import jax
import jax.numpy as jnp
from jax import lax
from jax.experimental import pallas as pl
from jax.experimental.pallas import tpu as pltpu
from jax.experimental.pallas import tpu_sc as plsc

_SCALING = 2.0
_BV = 4096
_CH = 128


def _fuse_body(w_ref, a_ref, b_ref, o_ref):
    o_ref[...] = w_ref[...] + _SCALING * lax.dot_general(
        a_ref[...], b_ref[...],
        dimension_numbers=(((0,), (1,)), ((), ())),
        preferred_element_type=jnp.float32)


def _fused_table(weight, lora_A, lora_B, interpret=False):
    V, D = weight.shape
    R = lora_A.shape[0]
    return pl.pallas_call(
        _fuse_body,
        grid=((V + _BV - 1) // _BV,),
        in_specs=[
            pl.BlockSpec((_BV, D), lambda i: (i, 0)),
            pl.BlockSpec((R, _BV), lambda i: (0, i)),
            pl.BlockSpec((D, R), lambda i: (0, 0)),
        ],
        out_specs=pl.BlockSpec((_BV, D), lambda i: (i, 0)),
        out_shape=jax.ShapeDtypeStruct((V, D), jnp.float32),
        interpret=interpret,
    )(weight, lora_A, lora_B)


_SBC = 2
_SB = _SBC * _CH


def _make_gather(n_tok, D):
    info = plsc.get_sparse_core_info()
    NC, NS = info.num_cores, info.num_subcores
    NW = NC * NS
    per_w = n_tok // NW
    n_ch = per_w // _CH
    n_sb = per_w // _SB
    mesh = plsc.VectorSubcoreMesh(core_axis_name="c", subcore_axis_name="s")

    @functools.partial(
        pl.kernel, mesh=mesh,
        out_type=jax.ShapeDtypeStruct((n_tok, D), jnp.float32),
        scratch_types=[
            pltpu.VMEM((n_ch, _CH), jnp.int32),
            pltpu.VMEM((2, _SB, D), jnp.float32),
        ] + [pltpu.SemaphoreType.DMA] * 4,
    )
    def gather_k(table_hbm, idx_hbm, out_hbm, idx_v, rows_v, *sems):
        gsem = sems[:2]
        wsem = sems[2:]
        wid = lax.axis_index("s") * NC + lax.axis_index("c")
        base = wid * per_w
        pltpu.sync_copy(idx_hbm.at[pl.ds(wid * n_ch, n_ch)], idx_v)

        def start_gathers(i, b):
            for q in range(_SBC):
                pltpu.async_copy(
                    table_hbm.at[idx_v.at[i * _SBC + q]],
                    rows_v.at[b].at[pl.ds(q * _CH, _CH)], gsem[b])

        def wait_gathers(i, b):
            for q in range(_SBC):
                pltpu.make_async_copy(
                    table_hbm.at[idx_v.at[i * _SBC + q]],
                    rows_v.at[b].at[pl.ds(q * _CH, _CH)], gsem[b]).wait()

        def start_wb(i, b):
            pltpu.async_copy(
                rows_v.at[b], out_hbm.at[pl.ds(base + i * _SB, _SB)], wsem[b])

        def wait_wb(b):
            pltpu.make_async_copy(
                rows_v.at[b], out_hbm.at[pl.ds(base, _SB)], wsem[b]).wait()

        start_gathers(0, 0)
        start_wb(1, 1)

        def outer(o, carry):
            for b in range(2):
                i = o * 2 + b
                wait_gathers(i, b)
                start_wb(i, b)

                @pl.when(i + 1 < n_sb)
                def _():
                    wait_wb(1 - b)
                    start_gathers(i + 1, 1 - b)
            return carry

        lax.fori_loop(0, n_sb // 2, outer, 0)
        wait_wb(0)
        wait_wb(1)

    return gather_k


def kernel(x, weight, lora_A, lora_B):
    B, L = x.shape
    V, D = weight.shape
    n_tok = B * L
    table = _fused_table(weight, lora_A, lora_B)
    idx2d = x.reshape(n_tok // _CH, _CH).astype(jnp.int32)
    out = _make_gather(n_tok, D)(table, idx2d)
    return out.reshape(B, L, D)


# speedup vs baseline: 1.8700x; 1.0133x over previous
import functools

import jax
import jax.numpy as jnp
from jax import lax
from jax.experimental import pallas as pl
from jax.experimental.pallas import tpu as pltpu
from jax.experimental.pallas import tpu_sc as plsc

_SCALING = 2.0
_BV = 8192
_CH = 128


def _fuse_body(w_ref, a_ref, b_ref, o_ref):
    o_ref[...] = w_ref[...] + _SCALING * lax.dot_general(
        a_ref[...], b_ref[...],
        dimension_numbers=(((0,), (1,)), ((), ())),
        preferred_element_type=jnp.float32)


def _fused_table(weight, lora_A, lora_B, interpret=False):
    V, D = weight.shape
    R = lora_A.shape[0]
    return pl.pallas_call(
        _fuse_body,
        grid=((V + _BV - 1) // _BV,),
        in_specs=[
            pl.BlockSpec((_BV, D), lambda i: (i, 0)),
            pl.BlockSpec((R, _BV), lambda i: (0, i)),
            pl.BlockSpec((D, R), lambda i: (0, 0)),
        ],
        out_specs=pl.BlockSpec((_BV, D), lambda i: (i, 0)),
        out_shape=jax.ShapeDtypeStruct((V, D), jnp.float32),
        interpret=interpret,
    )(weight, lora_A, lora_B)


_SBC = 2
_SB = _SBC * _CH


def _make_gather(n_tok, D):
    info = plsc.get_sparse_core_info()
    NC, NS = info.num_cores, info.num_subcores
    NW = NC * NS
    per_w = n_tok // NW
    n_ch = per_w // _CH
    n_sb = per_w // _SB
    mesh = plsc.VectorSubcoreMesh(core_axis_name="c", subcore_axis_name="s")

    @functools.partial(
        pl.kernel, mesh=mesh,
        out_type=jax.ShapeDtypeStruct((n_tok, D), jnp.float32),
        scratch_types=[
            pltpu.VMEM((n_ch, _CH), jnp.int32),
            pltpu.VMEM((2, _SB, D), jnp.float32),
        ] + [pltpu.SemaphoreType.DMA] * 4,
    )
    def gather_k(table_hbm, idx_hbm, out_hbm, idx_v, rows_v, *sems):
        gsem = sems[:2]
        wsem = sems[2:]
        wid = lax.axis_index("s") * NC + lax.axis_index("c")
        base = wid * per_w
        pltpu.sync_copy(idx_hbm.at[pl.ds(wid * n_ch, n_ch)], idx_v)

        def start_gathers(i, b):
            for q in range(_SBC):
                pltpu.async_copy(
                    table_hbm.at[idx_v.at[i * _SBC + q]],
                    rows_v.at[b].at[pl.ds(q * _CH, _CH)], gsem[b])

        def wait_gathers(i, b):
            for q in range(_SBC):
                pltpu.make_async_copy(
                    table_hbm.at[idx_v.at[i * _SBC + q]],
                    rows_v.at[b].at[pl.ds(q * _CH, _CH)], gsem[b]).wait()

        def start_wb(i, b):
            pltpu.async_copy(
                rows_v.at[b], out_hbm.at[pl.ds(base + i * _SB, _SB)], wsem[b])

        def wait_wb(b):
            pltpu.make_async_copy(
                rows_v.at[b], out_hbm.at[pl.ds(base, _SB)], wsem[b]).wait()

        start_gathers(0, 0)
        start_wb(1, 1)

        def outer(o, carry):
            for b in range(2):
                i = o * 2 + b
                wait_gathers(i, b)
                start_wb(i, b)

                @pl.when(i + 1 < n_sb)
                def _():
                    wait_wb(1 - b)
                    start_gathers(i + 1, 1 - b)
            return carry

        lax.fori_loop(0, n_sb // 2, outer, 0)
        wait_wb(0)
        wait_wb(1)

    return gather_k


def kernel(x, weight, lora_A, lora_B):
    B, L = x.shape
    V, D = weight.shape
    n_tok = B * L
    table = _fused_table(weight, lora_A, lora_B)
    idx2d = x.reshape(n_tok // _CH, _CH).astype(jnp.int32)
    out = _make_gather(n_tok, D)(table, idx2d)
    return out.reshape(B, L, D)


# speedup vs baseline: 1.8750x; 1.0027x over previous
import functools

import jax
import jax.numpy as jnp
from jax import lax
from jax.experimental import pallas as pl
from jax.experimental.pallas import tpu as pltpu
from jax.experimental.pallas import tpu_sc as plsc

_SCALING = 2.0
_BV = 12800
_CH = 128


def _fuse_body(w_ref, a_ref, b_ref, o_ref):
    o_ref[...] = w_ref[...] + _SCALING * lax.dot_general(
        a_ref[...], b_ref[...],
        dimension_numbers=(((0,), (1,)), ((), ())),
        preferred_element_type=jnp.float32)


def _fused_table(weight, lora_A, lora_B, interpret=False):
    V, D = weight.shape
    R = lora_A.shape[0]
    return pl.pallas_call(
        _fuse_body,
        grid=((V + _BV - 1) // _BV,),
        in_specs=[
            pl.BlockSpec((_BV, D), lambda i: (i, 0)),
            pl.BlockSpec((R, _BV), lambda i: (0, i)),
            pl.BlockSpec((D, R), lambda i: (0, 0)),
        ],
        out_specs=pl.BlockSpec((_BV, D), lambda i: (i, 0)),
        out_shape=jax.ShapeDtypeStruct((V, D), jnp.float32),
        interpret=interpret,
    )(weight, lora_A, lora_B)


_SBC = 2
_SB = _SBC * _CH


def _make_gather(n_tok, D):
    info = plsc.get_sparse_core_info()
    NC, NS = info.num_cores, info.num_subcores
    NW = NC * NS
    per_w = n_tok // NW
    n_ch = per_w // _CH
    n_sb = per_w // _SB
    mesh = plsc.VectorSubcoreMesh(core_axis_name="c", subcore_axis_name="s")

    @functools.partial(
        pl.kernel, mesh=mesh,
        out_type=jax.ShapeDtypeStruct((n_tok, D), jnp.float32),
        scratch_types=[
            pltpu.VMEM((n_ch, _CH), jnp.int32),
            pltpu.VMEM((2, _SB, D), jnp.float32),
        ] + [pltpu.SemaphoreType.DMA] * 4,
    )
    def gather_k(table_hbm, idx_hbm, out_hbm, idx_v, rows_v, *sems):
        gsem = sems[:2]
        wsem = sems[2:]
        wid = lax.axis_index("s") * NC + lax.axis_index("c")
        base = wid * per_w
        pltpu.sync_copy(idx_hbm.at[pl.ds(wid * n_ch, n_ch)], idx_v)

        def start_gathers(i, b):
            for q in range(_SBC):
                pltpu.async_copy(
                    table_hbm.at[idx_v.at[i * _SBC + q]],
                    rows_v.at[b].at[pl.ds(q * _CH, _CH)], gsem[b])

        def wait_gathers(i, b):
            for q in range(_SBC):
                pltpu.make_async_copy(
                    table_hbm.at[idx_v.at[i * _SBC + q]],
                    rows_v.at[b].at[pl.ds(q * _CH, _CH)], gsem[b]).wait()

        def start_wb(i, b):
            pltpu.async_copy(
                rows_v.at[b], out_hbm.at[pl.ds(base + i * _SB, _SB)], wsem[b])

        def wait_wb(b):
            pltpu.make_async_copy(
                rows_v.at[b], out_hbm.at[pl.ds(base, _SB)], wsem[b]).wait()

        start_gathers(0, 0)
        start_wb(1, 1)

        def outer(o, carry):
            for b in range(2):
                i = o * 2 + b
                wait_gathers(i, b)
                start_wb(i, b)

                @pl.when(i + 1 < n_sb)
                def _():
                    wait_wb(1 - b)
                    start_gathers(i + 1, 1 - b)
            return carry

        lax.fori_loop(0, n_sb // 2, outer, 0)
        wait_wb(0)
        wait_wb(1)

    return gather_k


def kernel(x, weight, lora_A, lora_B):
    B, L = x.shape
    V, D = weight.shape
    n_tok = B * L
    table = _fused_table(weight, lora_A, lora_B)
    idx2d = x.reshape(n_tok // _CH, _CH).astype(jnp.int32)
    out = _make_gather(n_tok, D)(table, idx2d)
    return out.reshape(B, L, D)


# speedup vs baseline: 1.8804x; 1.0028x over previous
import functools

import jax
import jax.numpy as jnp
from jax import lax
from jax.experimental import pallas as pl
from jax.experimental.pallas import tpu as pltpu
from jax.experimental.pallas import tpu_sc as plsc

_SCALING = 2.0
_BV = 16384
_CH = 128


def _fuse_body(w_ref, a_ref, b_ref, o_ref):
    o_ref[...] = w_ref[...] + _SCALING * lax.dot_general(
        a_ref[...], b_ref[...],
        dimension_numbers=(((0,), (1,)), ((), ())),
        preferred_element_type=jnp.float32)


def _fused_table(weight, lora_A, lora_B, interpret=False):
    V, D = weight.shape
    R = lora_A.shape[0]
    return pl.pallas_call(
        _fuse_body,
        grid=((V + _BV - 1) // _BV,),
        in_specs=[
            pl.BlockSpec((_BV, D), lambda i: (i, 0)),
            pl.BlockSpec((R, _BV), lambda i: (0, i)),
            pl.BlockSpec((D, R), lambda i: (0, 0)),
        ],
        out_specs=pl.BlockSpec((_BV, D), lambda i: (i, 0)),
        out_shape=jax.ShapeDtypeStruct((V, D), jnp.float32),
        interpret=interpret,
    )(weight, lora_A, lora_B)


_SBC = 2
_SB = _SBC * _CH


def _make_gather(n_tok, D):
    info = plsc.get_sparse_core_info()
    NC, NS = info.num_cores, info.num_subcores
    NW = NC * NS
    per_w = n_tok // NW
    n_ch = per_w // _CH
    n_sb = per_w // _SB
    mesh = plsc.VectorSubcoreMesh(core_axis_name="c", subcore_axis_name="s")

    @functools.partial(
        pl.kernel, mesh=mesh,
        out_type=jax.ShapeDtypeStruct((n_tok, D), jnp.float32),
        scratch_types=[
            pltpu.VMEM((n_ch, _CH), jnp.int32),
            pltpu.VMEM((2, _SB, D), jnp.float32),
        ] + [pltpu.SemaphoreType.DMA] * 4,
    )
    def gather_k(table_hbm, idx_hbm, out_hbm, idx_v, rows_v, *sems):
        gsem = sems[:2]
        wsem = sems[2:]
        wid = lax.axis_index("s") * NC + lax.axis_index("c")
        base = wid * per_w
        pltpu.sync_copy(idx_hbm.at[pl.ds(wid * n_ch, n_ch)], idx_v)

        def start_gathers(i, b):
            for q in range(_SBC):
                pltpu.async_copy(
                    table_hbm.at[idx_v.at[i * _SBC + q]],
                    rows_v.at[b].at[pl.ds(q * _CH, _CH)], gsem[b])

        def wait_gathers(i, b):
            for q in range(_SBC):
                pltpu.make_async_copy(
                    table_hbm.at[idx_v.at[i * _SBC + q]],
                    rows_v.at[b].at[pl.ds(q * _CH, _CH)], gsem[b]).wait()

        def start_wb(i, b):
            pltpu.async_copy(
                rows_v.at[b], out_hbm.at[pl.ds(base + i * _SB, _SB)], wsem[b])

        def wait_wb(b):
            pltpu.make_async_copy(
                rows_v.at[b], out_hbm.at[pl.ds(base, _SB)], wsem[b]).wait()

        start_gathers(0, 0)
        start_wb(1, 1)

        def outer(o, carry):
            for b in range(2):
                i = o * 2 + b
                wait_gathers(i, b)
                start_wb(i, b)

                @pl.when(i + 1 < n_sb)
                def _():
                    wait_wb(1 - b)
                    start_gathers(i + 1, 1 - b)
            return carry

        lax.fori_loop(0, n_sb // 2, outer, 0)
        wait_wb(0)
        wait_wb(1)

    return gather_k


def kernel(x, weight, lora_A, lora_B):
    B, L = x.shape
    V, D = weight.shape
    n_tok = B * L
    table = _fused_table(weight, lora_A, lora_B)
    idx2d = x.reshape(n_tok // _CH, _CH).astype(jnp.int32)
    out = _make_gather(n_tok, D)(table, idx2d)
    return out.reshape(B, L, D)
